# baseline (speedup 1.0000x reference)
import functools

import jax
import jax.numpy as jnp
from jax.experimental import pallas as pl
from jax.experimental.pallas import tpu as pltpu


def _rank_lt(cols, k):
    sel = []
    for i, ci in enumerate(cols):
        rank = None
        for j, cj in enumerate(cols):
            if j == i:
                continue
            beats = (cj > ci) if j > i else (cj >= ci)
            b = beats.astype(jnp.float32)
            rank = b if rank is None else rank + b
        sel.append(rank < k)
    return sel


def _router_comb(xb, gate_w, n_group, topk_group, top_k):
    logits = jax.lax.dot_general(
        xb, gate_w, (((1,), (1,)), ((), ())),
        preferred_element_type=jnp.float32)
    m = jnp.max(logits, axis=-1, keepdims=True)
    p = jnp.exp(logits - m)
    p = p / jnp.sum(p, axis=-1, keepdims=True)
    e_total = p.shape[-1]
    per_g = e_total // n_group
    pcols = [p[:, i:i + 1] for i in range(e_total)]
    gcols = []
    for g in range(n_group):
        gc = pcols[g * per_g]
        for r in range(1, per_g):
            gc = jnp.maximum(gc, pcols[g * per_g + r])
        gcols.append(gc)
    gsel = _rank_lt(gcols, topk_group)
    tcols = [jnp.where(gsel[i // per_g], pcols[i], 0.0) for i in range(e_total)]
    esel = _rank_lt(tcols, top_k)
    ccols = [jnp.where(esel[i], tcols[i], 0.0) for i in range(e_total)]
    return jnp.concatenate(ccols, axis=-1)


def _moe_body(x_ref, gate_ref, w1_ref, w2_ref, out_ref, comb_ref,
              *, n_routed, n_group, topk_group, top_k, bt, f):
    e = pl.program_id(0)
    t = pl.program_id(1)
    x32 = x_ref[...]

    @pl.when(e == 0)
    def _():
        comb_ref[pl.ds(t * bt, bt), :] = _router_comb(
            x32, gate_ref[...], n_group, topk_group, top_k)

    comb = comb_ref[pl.ds(t * bt, bt), :]
    e_idx = jnp.minimum(e, n_routed - 1)
    onehot = (jax.lax.broadcasted_iota(jnp.int32, (1, n_routed), 1)
              == e_idx).astype(jnp.float32)
    w_col = jnp.sum(comb * onehot, axis=-1, keepdims=True)
    w_col = jnp.where(e < n_routed, w_col, 1.0)

    xb = x32.astype(jnp.bfloat16)
    h = jax.lax.dot_general(
        xb, w1_ref[0], (((1,), (0,)), ((), ())),
        preferred_element_type=jnp.float32)
    hg = h[:, :f]
    hu = h[:, f:]
    a = (hg * jax.nn.sigmoid(hg) * hu).astype(jnp.bfloat16)
    o = jax.lax.dot_general(
        a, w2_ref[0], (((1,), (0,)), ((), ())),
        preferred_element_type=jnp.float32)
    o = o * w_col
    rows = pl.ds(t * bt, bt)

    @pl.when(e == 0)
    def _():
        out_ref[rows, :] = o

    @pl.when(e > 0)
    def _():
        out_ref[rows, :] = out_ref[rows, :] + o


def kernel(x, gate_weight, w1, w2, shared_w1, shared_w2):
    T, D = x.shape
    E, F2, _ = w1.shape
    F = F2 // 2
    FS = shared_w2.shape[1]
    NP = FS // F
    n_group, topk_group, top_k = 4, 2, 2
    BT = min(512, T)

    sg = shared_w1[:FS]
    su = shared_w1[FS:]
    pw1 = jnp.stack([
        jnp.concatenate([sg[i * F:(i + 1) * F], su[i * F:(i + 1) * F]], axis=0)
        for i in range(NP)])
    pw2 = jnp.stack([shared_w2[:, i * F:(i + 1) * F] for i in range(NP)])
    W1 = jnp.concatenate([w1, pw1], axis=0).astype(jnp.bfloat16).transpose(0, 2, 1)
    W2 = jnp.concatenate([w2, pw2], axis=0).astype(jnp.bfloat16).transpose(0, 2, 1)
    NE = E + NP

    body = functools.partial(
        _moe_body, n_routed=E, n_group=n_group, topk_group=topk_group,
        top_k=top_k, bt=BT, f=F)

    return pl.pallas_call(
        body,
        grid=(NE, T // BT),
        in_specs=[
            pl.BlockSpec((BT, D), lambda e, t: (t, 0)),
            pl.BlockSpec((E, D), lambda e, t: (0, 0)),
            pl.BlockSpec((1, D, F2), lambda e, t: (e, 0, 0)),
            pl.BlockSpec((1, F, D), lambda e, t: (e, 0, 0)),
        ],
        out_specs=pl.BlockSpec((T, D), lambda e, t: (0, 0)),
        out_shape=jax.ShapeDtypeStruct((T, D), jnp.float32),
        scratch_shapes=[pltpu.VMEM((T, E), jnp.float32)],
        compiler_params=pltpu.CompilerParams(
            dimension_semantics=("arbitrary", "arbitrary"),
        ),
    )(x, gate_weight, W1, W2)


# speedup vs baseline: 1.1235x; 1.1235x over previous
import functools

import jax
import jax.numpy as jnp
from jax.experimental import pallas as pl
from jax.experimental.pallas import tpu as pltpu


def _rank_lt(cols, k):
    sel = []
    for i, ci in enumerate(cols):
        rank = None
        for j, cj in enumerate(cols):
            if j == i:
                continue
            beats = (cj > ci) if j > i else (cj >= ci)
            b = beats.astype(jnp.float32)
            rank = b if rank is None else rank + b
        sel.append(rank < k)
    return sel


def _router_comb(xb, gate_w, n_group, topk_group, top_k):
    logits = jax.lax.dot_general(
        xb, gate_w, (((1,), (1,)), ((), ())),
        preferred_element_type=jnp.float32)
    m = jnp.max(logits, axis=-1, keepdims=True)
    p = jnp.exp(logits - m)
    p = p / jnp.sum(p, axis=-1, keepdims=True)
    e_total = p.shape[-1]
    per_g = e_total // n_group
    pcols = [p[:, i:i + 1] for i in range(e_total)]
    gcols = []
    for g in range(n_group):
        gc = pcols[g * per_g]
        for r in range(1, per_g):
            gc = jnp.maximum(gc, pcols[g * per_g + r])
        gcols.append(gc)
    gsel = _rank_lt(gcols, topk_group)
    tcols = [jnp.where(gsel[i // per_g], pcols[i], 0.0) for i in range(e_total)]
    esel = _rank_lt(tcols, top_k)
    ccols = [jnp.where(esel[i], tcols[i], 0.0) for i in range(e_total)]
    return jnp.concatenate(ccols, axis=-1)


def _moe_body(x_ref, gate_ref, w1_ref, w2_ref, out_ref, comb_ref,
              *, n_routed, n_group, topk_group, top_k, bt, f):
    e = pl.program_id(0)
    t = pl.program_id(1)
    x32 = x_ref[...]

    @pl.when(e == 0)
    def _():
        comb_ref[pl.ds(t * bt, bt), :] = _router_comb(
            x32, gate_ref[...], n_group, topk_group, top_k)

    comb = comb_ref[pl.ds(t * bt, bt), :]
    e_idx = jnp.minimum(e, n_routed - 1)
    onehot = (jax.lax.broadcasted_iota(jnp.int32, (1, n_routed), 1)
              == e_idx).astype(jnp.float32)
    w_col = jnp.sum(comb * onehot, axis=-1, keepdims=True)
    w_col = jnp.where(e < n_routed, w_col, 1.0)

    xb = x32.astype(jnp.bfloat16)
    h = jax.lax.dot_general(
        xb, w1_ref[0], (((1,), (1,)), ((), ())),
        preferred_element_type=jnp.float32)
    hg = h[:, :f]
    hu = h[:, f:]
    a = (hg * jax.nn.sigmoid(hg) * hu).astype(jnp.bfloat16)
    o = jax.lax.dot_general(
        a, w2_ref[0], (((1,), (1,)), ((), ())),
        preferred_element_type=jnp.float32)
    o = o * w_col
    rows = pl.ds(t * bt, bt)

    @pl.when(e == 0)
    def _():
        out_ref[rows, :] = o

    @pl.when(e > 0)
    def _():
        out_ref[rows, :] = out_ref[rows, :] + o


def kernel(x, gate_weight, w1, w2, shared_w1, shared_w2):
    T, D = x.shape
    E, F2, _ = w1.shape
    F = F2 // 2
    FS = shared_w2.shape[1]
    NP = FS // F
    n_group, topk_group, top_k = 4, 2, 2
    BT = min(512, T)

    sg = shared_w1[:FS]
    su = shared_w1[FS:]
    pw1 = jnp.stack([
        jnp.concatenate([sg[i * F:(i + 1) * F], su[i * F:(i + 1) * F]], axis=0)
        for i in range(NP)])
    pw2 = jnp.stack([shared_w2[:, i * F:(i + 1) * F] for i in range(NP)])
    W1 = jnp.concatenate([w1, pw1], axis=0).astype(jnp.bfloat16)
    W2 = jnp.concatenate([w2, pw2], axis=0).astype(jnp.bfloat16)
    NE = E + NP

    body = functools.partial(
        _moe_body, n_routed=E, n_group=n_group, topk_group=topk_group,
        top_k=top_k, bt=BT, f=F)

    return pl.pallas_call(
        body,
        grid=(NE, T // BT),
        in_specs=[
            pl.BlockSpec((BT, D), lambda e, t: (t, 0)),
            pl.BlockSpec((E, D), lambda e, t: (0, 0)),
            pl.BlockSpec((1, F2, D), lambda e, t: (e, 0, 0)),
            pl.BlockSpec((1, D, F), lambda e, t: (e, 0, 0)),
        ],
        out_specs=pl.BlockSpec((T, D), lambda e, t: (0, 0)),
        out_shape=jax.ShapeDtypeStruct((T, D), jnp.float32),
        scratch_shapes=[pltpu.VMEM((T, E), jnp.float32)],
        compiler_params=pltpu.CompilerParams(
            dimension_semantics=("arbitrary", "arbitrary"),
        ),
    )(x, gate_weight, W1, W2)


# speedup vs baseline: 1.1825x; 1.0525x over previous
import functools

import jax
import jax.numpy as jnp
from jax.experimental import pallas as pl
from jax.experimental.pallas import tpu as pltpu


def _rank_lt(rows, k):
    sel = []
    for i, ci in enumerate(rows):
        rank = None
        for j, cj in enumerate(rows):
            if j == i:
                continue
            beats = (cj > ci) if j > i else (cj >= ci)
            b = beats.astype(jnp.float32)
            rank = b if rank is None else rank + b
        sel.append(rank < k)
    return sel


def _router_body(x_ref, gate_ref, comb_ref, *, n_group, topk_group, top_k):
    lt = jax.lax.dot_general(
        gate_ref[...], x_ref[...], (((1,), (1,)), ((), ())),
        preferred_element_type=jnp.float32)
    m = jnp.max(lt, axis=0, keepdims=True)
    p = jnp.exp(lt - m)
    p = p / jnp.sum(p, axis=0, keepdims=True)
    e_total = p.shape[0]
    per_g = e_total // n_group
    prows = [p[i:i + 1, :] for i in range(e_total)]
    grows = []
    for g in range(n_group):
        gc = prows[g * per_g]
        for r in range(1, per_g):
            gc = jnp.maximum(gc, prows[g * per_g + r])
        grows.append(gc)
    gsel = _rank_lt(grows, topk_group)
    trows = [jnp.where(gsel[i // per_g], prows[i], 0.0) for i in range(e_total)]
    esel = _rank_lt(trows, top_k)
    crows = [jnp.where(esel[i], trows[i], 0.0) for i in range(e_total)]
    comb_t = jnp.concatenate(crows, axis=0)
    comb_ref[...] = comb_t.T


def _moe_body(x_ref, comb_ref, w1_ref, w2_ref, out_ref,
              *, n_routed, bt, f):
    e = pl.program_id(0)
    t = pl.program_id(1)
    x32 = x_ref[...]

    comb = comb_ref[...]
    e_idx = jnp.minimum(e, n_routed - 1)
    onehot = (jax.lax.broadcasted_iota(jnp.int32, (1, n_routed), 1)
              == e_idx).astype(jnp.float32)
    w_col = jnp.sum(comb * onehot, axis=-1, keepdims=True)
    w_col = jnp.where(e < n_routed, w_col, 1.0)

    xb = x32.astype(jnp.bfloat16)
    h = jax.lax.dot_general(
        xb, w1_ref[0], (((1,), (1,)), ((), ())),
        preferred_element_type=jnp.float32)
    hg = h[:, :f]
    hu = h[:, f:]
    a = (hg * jax.nn.sigmoid(hg) * hu).astype(jnp.bfloat16)
    o = jax.lax.dot_general(
        a, w2_ref[0], (((1,), (1,)), ((), ())),
        preferred_element_type=jnp.float32)
    o = o * w_col
    rows = pl.ds(t * bt, bt)

    @pl.when(e == 0)
    def _():
        out_ref[rows, :] = o

    @pl.when(e > 0)
    def _():
        out_ref[rows, :] = out_ref[rows, :] + o


def kernel(x, gate_weight, w1, w2, shared_w1, shared_w2):
    T, D = x.shape
    E, F2, _ = w1.shape
    F = F2 // 2
    FS = shared_w2.shape[1]
    NP = FS // F
    n_group, topk_group, top_k = 4, 2, 2
    BT = min(512, T)

    comb = pl.pallas_call(
        functools.partial(_router_body, n_group=n_group,
                          topk_group=topk_group, top_k=top_k),
        grid=(T // BT,),
        in_specs=[
            pl.BlockSpec((BT, D), lambda t: (t, 0)),
            pl.BlockSpec((E, D), lambda t: (0, 0)),
        ],
        out_specs=pl.BlockSpec((BT, E), lambda t: (t, 0)),
        out_shape=jax.ShapeDtypeStruct((T, E), jnp.float32),
    )(x, gate_weight)

    sg = shared_w1[:FS]
    su = shared_w1[FS:]
    pw1 = jnp.stack([
        jnp.concatenate([sg[i * F:(i + 1) * F], su[i * F:(i + 1) * F]], axis=0)
        for i in range(NP)])
    pw2 = jnp.stack([shared_w2[:, i * F:(i + 1) * F] for i in range(NP)])
    W1 = jnp.concatenate([w1, pw1], axis=0).astype(jnp.bfloat16)
    W2 = jnp.concatenate([w2, pw2], axis=0).astype(jnp.bfloat16)
    NE = E + NP

    body = functools.partial(_moe_body, n_routed=E, bt=BT, f=F)

    return pl.pallas_call(
        body,
        grid=(NE, T // BT),
        in_specs=[
            pl.BlockSpec((BT, D), lambda e, t: (t, 0)),
            pl.BlockSpec((BT, E), lambda e, t: (t, 0)),
            pl.BlockSpec((1, F2, D), lambda e, t: (e, 0, 0)),
            pl.BlockSpec((1, D, F), lambda e, t: (e, 0, 0)),
        ],
        out_specs=pl.BlockSpec((T, D), lambda e, t: (0, 0)),
        out_shape=jax.ShapeDtypeStruct((T, D), jnp.float32),
        compiler_params=pltpu.CompilerParams(
            dimension_semantics=("arbitrary", "arbitrary"),
        ),
    )(x, comb, W1, W2)
